# Initial kernel scaffold; baseline (speedup 1.0000x reference)
import functools

import jax
import jax.numpy as jnp
from jax import lax
from jax.experimental import pallas as pl
from jax.experimental.pallas import tpu as pltpu
from jax.experimental.pallas import tpu_sc as plsc

EMB = 64
B = 16384
L = 50

NC = 2
NS = 16
NW = NC * NS
BAGS_PER_W = B // NW
CHUNK = 16
N_CHUNKS = BAGS_PER_W // CHUNK
IDX_PER_CHUNK = CHUNK * L
VPR = EMB // 16


def _emb_body(x_hbm, tab_hbm, out_hbm, idx_v, rows_v, acc_v, sem):
    wid = lax.axis_index("s") * NC + lax.axis_index("c")
    bag0 = wid * BAGS_PER_W

    def chunk_body(c, carry):
        base_bag = bag0 + c * CHUNK
        pltpu.sync_copy(x_hbm.at[pl.ds(base_bag * L, IDX_PER_CHUNK)], idx_v)
        pltpu.async_copy(tab_hbm.at[idx_v], rows_v, sem).wait()

        def bag_body(j, carry2):
            def row_body(i, accs):
                r = j * L + i
                return tuple(a + rows_v[r, pl.ds(16 * k, 16)]
                             for k, a in enumerate(accs))

            zero = tuple(jnp.zeros((16,), jnp.float32) for _ in range(VPR))
            accs = lax.fori_loop(0, L, row_body, zero)
            for k in range(VPR):
                acc_v[j, pl.ds(16 * k, 16)] = accs[k] * (1.0 / L)
            return carry2

        lax.fori_loop(0, CHUNK, bag_body, 0)
        pltpu.sync_copy(acc_v, out_hbm.at[pl.ds(base_bag, CHUNK)])
        return carry

    lax.fori_loop(0, N_CHUNKS, chunk_body, 0)


_embed_bag = functools.partial(
    pl.kernel,
    mesh=plsc.VectorSubcoreMesh(core_axis_name="c", subcore_axis_name="s"),
    out_type=jax.ShapeDtypeStruct((B, EMB), jnp.float32),
    scratch_types=[
        pltpu.VMEM((IDX_PER_CHUNK,), jnp.int32),
        pltpu.VMEM((IDX_PER_CHUNK, EMB), jnp.float32),
        pltpu.VMEM((CHUNK, EMB), jnp.float32),
        pltpu.SemaphoreType.DMA,
    ],
)(_emb_body)


def _mlp_body(e_ref, w1_ref, b1_ref, w2_ref, b2_ref, o_ref):
    h = jnp.dot(e_ref[...], w1_ref[...],
                preferred_element_type=jnp.float32) + b1_ref[...]
    h = jnp.maximum(h, 0.0)
    z = jnp.sum(h * w2_ref[...], axis=1, keepdims=True) + b2_ref[...]
    o_ref[...] = 1.0 / (1.0 + jnp.exp(-z))


_N_BLOCKS = 8
_BLK = B // _N_BLOCKS

_mlp = pl.pallas_call(
    _mlp_body,
    grid=(_N_BLOCKS,),
    in_specs=[
        pl.BlockSpec((_BLK, EMB), lambda i: (i, 0)),
        pl.BlockSpec((EMB, 128), lambda i: (0, 0)),
        pl.BlockSpec((1, 128), lambda i: (0, 0)),
        pl.BlockSpec((1, 128), lambda i: (0, 0)),
        pl.BlockSpec((1, 1), lambda i: (0, 0)),
    ],
    out_specs=pl.BlockSpec((_BLK, 1), lambda i: (i, 0)),
    out_shape=jax.ShapeDtypeStruct((B, 1), jnp.float32),
)


def kernel(x, emb_table, W1, b1, W2, b2):
    embeds = _embed_bag(x.reshape(-1), emb_table)
    return _mlp(embeds, W1, b1.reshape(1, 128), W2.reshape(1, 128),
                b2.reshape(1, 1))


# speedup vs baseline: 2.4232x; 2.4232x over previous
import functools

import jax
import jax.numpy as jnp
from jax import lax
from jax.experimental import pallas as pl
from jax.experimental.pallas import tpu as pltpu
from jax.experimental.pallas import tpu_sc as plsc

EMB = 64
B = 16384
L = 50

NC = 2
NS = 16
NW = NC * NS
BAGS_PER_W = B // NW
CHUNK = 16
N_CHUNKS = BAGS_PER_W // CHUNK
IDX_PER_CHUNK = CHUNK * L
VPR = EMB // 16


def _emb_body(x_hbm, tab_hbm, out_hbm, idx_v, rows_v, acc_v, sem):
    wid = lax.axis_index("s") * NC + lax.axis_index("c")
    bag0 = wid * BAGS_PER_W

    def chunk_body(c, carry):
        base_bag = bag0 + c * CHUNK
        pltpu.sync_copy(x_hbm.at[pl.ds(base_bag * L, IDX_PER_CHUNK)], idx_v)
        pltpu.async_copy(tab_hbm.at[idx_v], rows_v, sem).wait()

        def bag_body(j, carry2):
            def row_body(i, accs):
                r = j * L + i
                return tuple(a + rows_v[r, pl.ds(16 * k, 16)]
                             for k, a in enumerate(accs))

            zero = tuple(jnp.zeros((16,), jnp.float32) for _ in range(VPR))
            accs = lax.fori_loop(0, L, row_body, zero)
            for k in range(VPR):
                acc_v[j, pl.ds(16 * k, 16)] = accs[k] * (1.0 / L)
            return carry2

        lax.fori_loop(0, CHUNK, bag_body, 0)
        pltpu.sync_copy(acc_v, out_hbm.at[pl.ds(base_bag, CHUNK)])
        return carry

    lax.fori_loop(0, N_CHUNKS, chunk_body, 0)


_embed_bag = functools.partial(
    pl.kernel,
    mesh=plsc.VectorSubcoreMesh(core_axis_name="c", subcore_axis_name="s"),
    out_type=jax.ShapeDtypeStruct((B, EMB), jnp.float32),
    scratch_types=[
        pltpu.VMEM((IDX_PER_CHUNK,), jnp.int32),
        pltpu.VMEM((IDX_PER_CHUNK, EMB), jnp.float32),
        pltpu.VMEM((CHUNK, EMB), jnp.float32),
        pltpu.SemaphoreType.DMA,
    ],
    compiler_params=pltpu.CompilerParams(use_tc_tiling_on_sc=False),
)(_emb_body)


def _mlp_body(e_ref, w1_ref, b1_ref, w2_ref, b2_ref, o_ref):
    h = jnp.dot(e_ref[...], w1_ref[...],
                preferred_element_type=jnp.float32) + b1_ref[...]
    h = jnp.maximum(h, 0.0)
    z = jnp.sum(h * w2_ref[...], axis=1, keepdims=True) + b2_ref[...]
    o_ref[...] = 1.0 / (1.0 + jnp.exp(-z))


_N_BLOCKS = 8
_BLK = B // _N_BLOCKS

_mlp = pl.pallas_call(
    _mlp_body,
    grid=(_N_BLOCKS,),
    in_specs=[
        pl.BlockSpec((_BLK, EMB), lambda i: (i, 0)),
        pl.BlockSpec((EMB, 128), lambda i: (0, 0)),
        pl.BlockSpec((1, 128), lambda i: (0, 0)),
        pl.BlockSpec((1, 128), lambda i: (0, 0)),
        pl.BlockSpec((1, 1), lambda i: (0, 0)),
    ],
    out_specs=pl.BlockSpec((_BLK, 1), lambda i: (i, 0)),
    out_shape=jax.ShapeDtypeStruct((B, 1), jnp.float32),
)


def kernel(x, emb_table, W1, b1, W2, b2):
    embeds = _embed_bag(x.reshape(-1), emb_table)
    return _mlp(embeds, W1, b1.reshape(1, 128), W2.reshape(1, 128),
                b2.reshape(1, 1))


# speedup vs baseline: 2.7680x; 1.1423x over previous
import functools

import jax
import jax.numpy as jnp
from jax import lax
from jax.experimental import pallas as pl
from jax.experimental.pallas import tpu as pltpu
from jax.experimental.pallas import tpu_sc as plsc

EMB = 64
B = 16384
L = 50

NC = 2
NS = 16
NW = NC * NS
BAGS_PER_W = B // NW
CHUNK = 16
N_CHUNKS = BAGS_PER_W // CHUNK
IDX_PER_CHUNK = CHUNK * L
VPR = EMB // 16


def _emb_body(x_hbm, tab_hbm, out_hbm, idx_v, rows0_v, rows1_v, acc_v,
              sem0, sem1):
    wid = lax.axis_index("s") * NC + lax.axis_index("c")
    bag0 = wid * BAGS_PER_W
    chunk0 = wid * N_CHUNKS

    pltpu.sync_copy(x_hbm.at[pl.ds(chunk0, N_CHUNKS)], idx_v)

    rows = (rows0_v, rows1_v)
    sems = (sem0, sem1)

    def start(c, b):
        return pltpu.async_copy(tab_hbm.at[idx_v.at[c]], rows[b], sems[b])

    def reduce_chunk(c, b):
        rows_v = rows[b]
        base_bag = bag0 + c * CHUNK

        def bag_body(j, carry2):
            base = j * L
            accs = tuple(rows_v[base, pl.ds(16 * k, 16)]
                         for k in range(VPR))
            for i in range(1, L):
                accs = tuple(a + rows_v[base + i, pl.ds(16 * k, 16)]
                             for k, a in enumerate(accs))
            for k in range(VPR):
                acc_v[j, pl.ds(16 * k, 16)] = accs[k] * (1.0 / L)
            return carry2

        lax.fori_loop(0, CHUNK, bag_body, 0)
        pltpu.sync_copy(acc_v, out_hbm.at[pl.ds(base_bag, CHUNK)])

    start(0, 0)

    def pair_body(p, carry):
        c0 = 2 * p
        start(c0 + 1, 1)
        pltpu.make_async_copy(tab_hbm.at[idx_v.at[c0]], rows[0],
                              sems[0]).wait()
        reduce_chunk(c0, 0)

        @pl.when(c0 + 2 < N_CHUNKS)
        def _():
            start(c0 + 2, 0)

        pltpu.make_async_copy(tab_hbm.at[idx_v.at[c0 + 1]], rows[1],
                              sems[1]).wait()
        reduce_chunk(c0 + 1, 1)
        return carry

    lax.fori_loop(0, N_CHUNKS // 2, pair_body, 0)


_embed_bag = functools.partial(
    pl.kernel,
    mesh=plsc.VectorSubcoreMesh(core_axis_name="c", subcore_axis_name="s"),
    out_type=jax.ShapeDtypeStruct((B, EMB), jnp.float32),
    scratch_types=[
        pltpu.VMEM((N_CHUNKS, IDX_PER_CHUNK), jnp.int32),
        pltpu.VMEM((IDX_PER_CHUNK, EMB), jnp.float32),
        pltpu.VMEM((IDX_PER_CHUNK, EMB), jnp.float32),
        pltpu.VMEM((CHUNK, EMB), jnp.float32),
        pltpu.SemaphoreType.DMA,
        pltpu.SemaphoreType.DMA,
    ],
    compiler_params=pltpu.CompilerParams(use_tc_tiling_on_sc=False),
)(_emb_body)


def _mlp_body(e_ref, w1_ref, b1_ref, w2_ref, b2_ref, o_ref):
    h = jnp.dot(e_ref[...], w1_ref[...],
                preferred_element_type=jnp.float32) + b1_ref[...]
    h = jnp.maximum(h, 0.0)
    z = jnp.sum(h * w2_ref[...], axis=1, keepdims=True) + b2_ref[...]
    o_ref[...] = 1.0 / (1.0 + jnp.exp(-z))


_N_BLOCKS = 8
_BLK = B // _N_BLOCKS

_mlp = pl.pallas_call(
    _mlp_body,
    grid=(_N_BLOCKS,),
    in_specs=[
        pl.BlockSpec((_BLK, EMB), lambda i: (i, 0)),
        pl.BlockSpec((EMB, 128), lambda i: (0, 0)),
        pl.BlockSpec((1, 128), lambda i: (0, 0)),
        pl.BlockSpec((1, 128), lambda i: (0, 0)),
        pl.BlockSpec((1, 1), lambda i: (0, 0)),
    ],
    out_specs=pl.BlockSpec((_BLK, 1), lambda i: (i, 0)),
    out_shape=jax.ShapeDtypeStruct((B, 1), jnp.float32),
)


def kernel(x, emb_table, W1, b1, W2, b2):
    embeds = _embed_bag(x.reshape(B // CHUNK, IDX_PER_CHUNK), emb_table)
    return _mlp(embeds, W1, b1.reshape(1, 128), W2.reshape(1, 128),
                b2.reshape(1, 1))
